# baseline (speedup 1.0000x reference)
import functools

import jax
import jax.numpy as jnp
from jax import lax
from jax.experimental import pallas as pl
from jax.experimental.pallas import tpu as pltpu
from jax.experimental.pallas import tpu_sc as plsc

BATCH = 1024
VOCAB = 100000
VS = 48000
NQ = 4
QV = VS // NQ
VC = 200
NCH = QV // VC
LANES = 16
JV = 128 // LANES
BV = 2000
TCN = (VOCAB - VS) // BV

_mesh = plsc.VectorSubcoreMesh(core_axis_name="c", subcore_axis_name="s")


@functools.partial(
    pl.kernel,
    mesh=_mesh,
    compiler_params=pltpu.CompilerParams(needs_layout_passes=False),
    out_type=(
        jax.ShapeDtypeStruct((BATCH,), jnp.float32),
        jax.ShapeDtypeStruct((BATCH,), jnp.int32),
    ),
    scratch_types=[
        pltpu.VMEM((VC, 128), jnp.float32),
        pltpu.VMEM((VC, 128), jnp.float32),
        pltpu.VMEM((VC, 128), jnp.float32),
        pltpu.VMEM((128,), jnp.float32),
        pltpu.VMEM((128,), jnp.int32),
        pltpu.VMEM((128,), jnp.float32),
        pltpu.VMEM((128,), jnp.int32),
        pltpu.VMEM_SHARED((16, 128), jnp.float32),
        pltpu.VMEM_SHARED((16, 128), jnp.int32),
        pltpu.SemaphoreType.DMA,
        pltpu.SemaphoreType.DMA,
        pltpu.SemaphoreType.DMA,
    ],
)
def _sc_scan(inpt_hbm, pm_hbm, pi_hbm,
             buf0, buf1, buf2, m_v, i_v, tm_v, ti_v,
             shm_m, shm_i, sem0, sem1, sem2):
    cid = lax.axis_index("c")
    sid = lax.axis_index("s")
    col = cid * 4 + sid % 4
    q = sid // 4
    b0 = col * 128
    v0 = q * QV
    bufs = (buf0, buf1, buf2)
    sems = (sem0, sem1, sem2)
    ninf = jnp.full((LANES,), -jnp.inf, jnp.float32)
    zero = jnp.zeros((LANES,), jnp.int32)

    def dma(c, par):
        vs = pl.multiple_of(v0 + c * VC, 8)
        bs = pl.multiple_of(b0, 128)
        return pltpu.make_async_copy(
            inpt_hbm.at[pl.ds(vs, VC), pl.ds(bs, 128)], bufs[par], sems[par])

    dma(0, 0).start()
    dma(1, 1).start()
    dma(2, 2).start()

    def scan_chunk(c, par, acc):
        bref = bufs[par]

        def vrow(t, acc2):
            accl = list(acc2)
            for k in range(2):
                v = t * 2 + k
                gv = v0 + c * VC + v
                gvv = jnp.full((LANES,), gv, jnp.int32)
                for j in range(JV):
                    x = bref[v, pl.ds(j * LANES, LANES)]
                    better = x > accl[j]
                    accl[j] = jnp.where(better, x, accl[j])
                    accl[JV + j] = jnp.where(better, gvv, accl[JV + j])
            return tuple(accl)

        return lax.fori_loop(0, VC // 2, vrow, acc)

    assert NCH % 3 == 0

    def chunk_trio(cc, acc):
        for par in (0, 1, 2):
            c = cc * 3 + par
            dma(c, par).wait()
            acc = scan_chunk(c, par, acc)

            @pl.when(c + 3 < NCH)
            def _(c=c, par=par):
                dma(c + 3, par).start()

        return acc

    acc = lax.fori_loop(0, NCH // 3, chunk_trio,
                        (ninf,) * JV + (zero,) * JV)

    for j in range(JV):
        m_v[pl.ds(j * LANES, LANES)] = acc[j]
        i_v[pl.ds(j * LANES, LANES)] = acc[JV + j]
    pltpu.sync_copy(m_v, shm_m.at[sid])
    pltpu.sync_copy(i_v, shm_i.at[sid])
    plsc.subcore_barrier()

    @pl.when(sid < 4)
    def _():
        for p in range(1, NQ):
            pltpu.sync_copy(shm_m.at[sid + p * 4], tm_v)
            pltpu.sync_copy(shm_i.at[sid + p * 4], ti_v)
            for j in range(JV):
                sl = pl.ds(j * LANES, LANES)
                m1, i1 = m_v[sl], i_v[sl]
                m2, i2 = tm_v[sl], ti_v[sl]
                take2 = (m2 > m1) | ((m2 == m1) & (i2 < i1))
                m_v[sl] = jnp.where(take2, m2, m1)
                i_v[sl] = jnp.where(take2, i2, i1)
        bs = pl.multiple_of(b0, 128)
        pltpu.sync_copy(m_v, pm_hbm.at[pl.ds(bs, 128)])
        pltpu.sync_copy(i_v, pi_hbm.at[pl.ds(bs, 128)])


def _tc_body(x_ref, om_ref, oi_ref, rm, ri):
    step = pl.program_id(0)

    @pl.when(step == 0)
    def _():
        rm[...] = jnp.full((8, BATCH), -jnp.inf, jnp.float32)
        ri[...] = jnp.zeros((8, BATCH), jnp.int32)

    iota8 = lax.broadcasted_iota(jnp.int32, (8, BATCH), 0)
    m = rm[...]
    idx = ri[...]
    for k in range(BV // 8):
        x = x_ref[pl.ds(k * 8, 8), :]
        gv = iota8 + (VS + step * BV + k * 8)
        better = x > m
        m = jnp.where(better, x, m)
        idx = jnp.where(better, gv, idx)
    rm[...] = m
    ri[...] = idx
    om_ref[...] = m
    oi_ref[...] = idx


_tc_scan = pl.pallas_call(
    _tc_body,
    grid=(TCN,),
    in_specs=[pl.BlockSpec((BV, BATCH), lambda i: (i + VS // BV, 0))],
    out_specs=(pl.BlockSpec((8, BATCH), lambda i: (0, 0)),
               pl.BlockSpec((8, BATCH), lambda i: (0, 0))),
    out_shape=(jax.ShapeDtypeStruct((8, BATCH), jnp.float32),
               jax.ShapeDtypeStruct((8, BATCH), jnp.int32)),
    scratch_shapes=[pltpu.VMEM((8, BATCH), jnp.float32),
                    pltpu.VMEM((8, BATCH), jnp.int32)],
)


@functools.partial(
    pl.kernel,
    mesh=_mesh,
    compiler_params=pltpu.CompilerParams(needs_layout_passes=False),
    out_type=(
        jax.ShapeDtypeStruct((BATCH,), jnp.float32),
        jax.ShapeDtypeStruct((BATCH,), jnp.float32),
    ),
    scratch_types=[
        pltpu.VMEM((128,), jnp.float32),
        pltpu.VMEM((128,), jnp.int32),
        pltpu.VMEM((8, 128), jnp.float32),
        pltpu.VMEM((8, 128), jnp.int32),
        pltpu.VMEM((128,), jnp.float32),
        pltpu.SemaphoreType.DMA,
    ],
)
def _merge(pm_hbm, pi_hbm, tcm_hbm, tci_hbm, tab_hbm, cast_hbm, maxp_hbm,
           m_v, i_v, tm_v, ti_v, cast_v, gsem):
    cid = lax.axis_index("c")
    sid = lax.axis_index("s")

    @pl.when(sid < 4)
    def _():
        col = cid * 4 + sid
        bs = pl.multiple_of(col * 128, 128)
        pltpu.sync_copy(pm_hbm.at[pl.ds(bs, 128)], m_v)
        pltpu.sync_copy(pi_hbm.at[pl.ds(bs, 128)], i_v)
        pltpu.sync_copy(tcm_hbm.at[:, pl.ds(bs, 128)], tm_v)
        pltpu.sync_copy(tci_hbm.at[:, pl.ds(bs, 128)], ti_v)
        for j in range(JV):
            sl = pl.ds(j * LANES, LANES)
            m1, i1 = m_v[sl], i_v[sl]
            for r in range(8):
                m2 = tm_v[r, sl]
                i2 = ti_v[r, sl]
                take2 = (m2 > m1) | ((m2 == m1) & (i2 < i1))
                m1 = jnp.where(take2, m2, m1)
                i1 = jnp.where(take2, i2, i1)
            m_v[sl] = m1
            i_v[sl] = i1
        g = pltpu.make_async_copy(tab_hbm.at[i_v], cast_v, gsem)
        g.start()
        g.wait()
        pltpu.sync_copy(cast_v, cast_hbm.at[pl.ds(bs, 128)])
        pltpu.sync_copy(m_v, maxp_hbm.at[pl.ds(bs, 128)])


def kernel(input, table_values):
    inpt = input.T
    sm, si = _sc_scan(inpt)
    tm, ti = _tc_scan(inpt)
    return _merge(sm, si, tm, ti, table_values)


# speedup vs baseline: 1.0190x; 1.0190x over previous
import functools

import jax
import jax.numpy as jnp
from jax import lax
from jax.experimental import pallas as pl
from jax.experimental.pallas import tpu as pltpu
from jax.experimental.pallas import tpu_sc as plsc

BATCH = 1024
VOCAB = 100000
VS = 48000
NQ = 4
QV = VS // NQ
VC = 200
NCH = QV // VC
LANES = 16
JV = 128 // LANES
BV = 2000
TCN = (VOCAB - VS) // BV

_mesh = plsc.VectorSubcoreMesh(core_axis_name="c", subcore_axis_name="s")


@functools.partial(
    pl.kernel,
    mesh=_mesh,
    compiler_params=pltpu.CompilerParams(needs_layout_passes=False),
    out_type=(
        jax.ShapeDtypeStruct((BATCH,), jnp.float32),
        jax.ShapeDtypeStruct((BATCH,), jnp.int32),
    ),
    scratch_types=[
        pltpu.VMEM((VC, 128), jnp.float32),
        pltpu.VMEM((VC, 128), jnp.float32),
        pltpu.VMEM((128,), jnp.float32),
        pltpu.VMEM((128,), jnp.int32),
        pltpu.VMEM((128,), jnp.float32),
        pltpu.VMEM((128,), jnp.int32),
        pltpu.VMEM_SHARED((16, 128), jnp.float32),
        pltpu.VMEM_SHARED((16, 128), jnp.int32),
        pltpu.SemaphoreType.DMA,
        pltpu.SemaphoreType.DMA,
    ],
)
def _sc_scan(inpt_hbm, pm_hbm, pi_hbm,
             buf0, buf1, m_v, i_v, tm_v, ti_v,
             shm_m, shm_i, sem0, sem1):
    cid = lax.axis_index("c")
    sid = lax.axis_index("s")
    col = cid * 4 + sid % 4
    q = sid // 4
    b0 = col * 128
    v0 = q * QV
    bufs = (buf0, buf1)
    sems = (sem0, sem1)
    ninf = jnp.full((LANES,), -jnp.inf, jnp.float32)
    zero = jnp.zeros((LANES,), jnp.int32)

    def dma(c, par):
        vs = pl.multiple_of(v0 + c * VC, 8)
        bs = pl.multiple_of(b0, 128)
        return pltpu.make_async_copy(
            inpt_hbm.at[pl.ds(vs, VC), pl.ds(bs, 128)], bufs[par], sems[par])

    dma(0, 0).start()
    dma(1, 1).start()

    def scan_chunk(c, par, acc):
        bref = bufs[par]

        def vrow(t, acc2):
            accl = list(acc2)
            for k in range(2):
                v = t * 2 + k
                gv = v0 + c * VC + v
                gvv = jnp.full((LANES,), gv, jnp.int32)
                for j in range(JV):
                    x = bref[v, pl.ds(j * LANES, LANES)]
                    better = x > accl[j]
                    accl[j] = jnp.where(better, x, accl[j])
                    accl[JV + j] = jnp.where(better, gvv, accl[JV + j])
            return tuple(accl)

        return lax.fori_loop(0, VC // 2, vrow, acc)

    def chunk_pair(cc, acc):
        for par in (0, 1):
            c = cc * 2 + par
            dma(c, par).wait()
            acc = scan_chunk(c, par, acc)

            @pl.when(c + 2 < NCH)
            def _(c=c, par=par):
                dma(c + 2, par).start()

        return acc

    acc = lax.fori_loop(0, NCH // 2, chunk_pair,
                        (ninf,) * JV + (zero,) * JV)
    if NCH % 2 == 1:
        dma(NCH - 1, 0).wait()
        acc = scan_chunk(NCH - 1, 0, acc)

    for j in range(JV):
        m_v[pl.ds(j * LANES, LANES)] = acc[j]
        i_v[pl.ds(j * LANES, LANES)] = acc[JV + j]
    pltpu.sync_copy(m_v, shm_m.at[sid])
    pltpu.sync_copy(i_v, shm_i.at[sid])
    plsc.subcore_barrier()

    @pl.when(sid < 4)
    def _():
        for p in range(1, NQ):
            pltpu.sync_copy(shm_m.at[sid + p * 4], tm_v)
            pltpu.sync_copy(shm_i.at[sid + p * 4], ti_v)
            for j in range(JV):
                sl = pl.ds(j * LANES, LANES)
                m1, i1 = m_v[sl], i_v[sl]
                m2, i2 = tm_v[sl], ti_v[sl]
                take2 = (m2 > m1) | ((m2 == m1) & (i2 < i1))
                m_v[sl] = jnp.where(take2, m2, m1)
                i_v[sl] = jnp.where(take2, i2, i1)
        bs = pl.multiple_of(b0, 128)
        pltpu.sync_copy(m_v, pm_hbm.at[pl.ds(bs, 128)])
        pltpu.sync_copy(i_v, pi_hbm.at[pl.ds(bs, 128)])


def _tc_body(x_ref, om_ref, oi_ref, rm, ri):
    step = pl.program_id(0)

    @pl.when(step == 0)
    def _():
        rm[...] = jnp.full((8, BATCH), -jnp.inf, jnp.float32)
        ri[...] = jnp.zeros((8, BATCH), jnp.int32)

    iota8 = lax.broadcasted_iota(jnp.int32, (8, BATCH), 0)
    m = rm[...]
    idx = ri[...]
    for k in range(BV // 8):
        x = x_ref[pl.ds(k * 8, 8), :]
        gv = iota8 + (VS + step * BV + k * 8)
        better = x > m
        m = jnp.where(better, x, m)
        idx = jnp.where(better, gv, idx)
    rm[...] = m
    ri[...] = idx
    om_ref[...] = m
    oi_ref[...] = idx


_tc_scan = pl.pallas_call(
    _tc_body,
    grid=(TCN,),
    in_specs=[pl.BlockSpec((BV, BATCH), lambda i: (i + VS // BV, 0))],
    out_specs=(pl.BlockSpec((8, BATCH), lambda i: (0, 0)),
               pl.BlockSpec((8, BATCH), lambda i: (0, 0))),
    out_shape=(jax.ShapeDtypeStruct((8, BATCH), jnp.float32),
               jax.ShapeDtypeStruct((8, BATCH), jnp.int32)),
    scratch_shapes=[pltpu.VMEM((8, BATCH), jnp.float32),
                    pltpu.VMEM((8, BATCH), jnp.int32)],
)


@functools.partial(
    pl.kernel,
    mesh=_mesh,
    compiler_params=pltpu.CompilerParams(needs_layout_passes=False),
    out_type=(
        jax.ShapeDtypeStruct((BATCH,), jnp.float32),
        jax.ShapeDtypeStruct((BATCH,), jnp.float32),
    ),
    scratch_types=[
        pltpu.VMEM((128,), jnp.float32),
        pltpu.VMEM((128,), jnp.int32),
        pltpu.VMEM((8, 128), jnp.float32),
        pltpu.VMEM((8, 128), jnp.int32),
        pltpu.VMEM((128,), jnp.float32),
        pltpu.SemaphoreType.DMA,
    ],
)
def _merge(pm_hbm, pi_hbm, tcm_hbm, tci_hbm, tab_hbm, cast_hbm, maxp_hbm,
           m_v, i_v, tm_v, ti_v, cast_v, gsem):
    cid = lax.axis_index("c")
    sid = lax.axis_index("s")

    @pl.when(sid < 4)
    def _():
        col = cid * 4 + sid
        bs = pl.multiple_of(col * 128, 128)
        pltpu.sync_copy(pm_hbm.at[pl.ds(bs, 128)], m_v)
        pltpu.sync_copy(pi_hbm.at[pl.ds(bs, 128)], i_v)
        pltpu.sync_copy(tcm_hbm.at[:, pl.ds(bs, 128)], tm_v)
        pltpu.sync_copy(tci_hbm.at[:, pl.ds(bs, 128)], ti_v)
        for j in range(JV):
            sl = pl.ds(j * LANES, LANES)
            m1, i1 = m_v[sl], i_v[sl]
            for r in range(8):
                m2 = tm_v[r, sl]
                i2 = ti_v[r, sl]
                take2 = (m2 > m1) | ((m2 == m1) & (i2 < i1))
                m1 = jnp.where(take2, m2, m1)
                i1 = jnp.where(take2, i2, i1)
            m_v[sl] = m1
            i_v[sl] = i1
        g = pltpu.make_async_copy(tab_hbm.at[i_v], cast_v, gsem)
        g.start()
        g.wait()
        pltpu.sync_copy(cast_v, cast_hbm.at[pl.ds(bs, 128)])
        pltpu.sync_copy(m_v, maxp_hbm.at[pl.ds(bs, 128)])


def kernel(input, table_values):
    inpt = input.T
    sm, si = _sc_scan(inpt)
    tm, ti = _tc_scan(inpt)
    return _merge(sm, si, tm, ti, table_values)
